# baseline (speedup 1.0000x reference)
import jax
import jax.numpy as jnp
from jax.experimental import pallas as pl
from jax.experimental.pallas import tpu as pltpu

_N = 4096
_H = 64
_NCLS = 10
_IB = 256
_NI = _N // _IB
_JB = 2048
_NJ = _N // _JB

_TDIMS = (((0,), (0,)), ((), ()))
_RTDIMS = (((1,), (1,)), ((), ()))


def _gcn_kernel(A_ref, x_ref, W1t_ref, W2_ref, W3_ref,
                b1_ref, b2_ref, b3_ref, LWt_ref, lb_ref,
                out_ref, Abig_scr, agg1_scr, Bt_scr, Ct_scr,
                h1t_scr, h2t_scr):
    t = pl.program_id(0)

    @pl.when(t < _NI)
    def _():
        rows = pl.ds(t * _IB, _IB)
        Abf = A_ref[...].astype(jnp.bfloat16)
        Abig_scr[rows, :] = Abf
        Bc = jax.lax.dot_general(
            W1t_ref[...], x_ref[...], _RTDIMS,
            preferred_element_type=jnp.float32)
        contrib = jnp.dot(Bc.astype(jnp.bfloat16), Abf,
                          preferred_element_type=jnp.float32)
        agg1_scr[...] = jnp.where(t == 0, contrib,
                                  agg1_scr[...] + contrib)

    @pl.when(t == _NI - 1)
    def _():
        x1t = jnp.maximum(agg1_scr[...] + b1_ref[...].reshape(_H, 1), 0.0)
        h1t_scr[...] = x1t
        Ct_scr[...] = jax.lax.dot_general(
            W2_ref[...], x1t, _TDIMS,
            preferred_element_type=jnp.float32).astype(jnp.bfloat16)

    @pl.when(jnp.logical_and(t >= _NI, t < _NI + _NJ))
    def _():
        cols = pl.ds((t - _NI) * _JB, _JB)
        agg_t = jnp.dot(Ct_scr[...], Abig_scr[:, cols],
                        preferred_element_type=jnp.float32)
        h2t_scr[:, cols] = jnp.maximum(agg_t + b2_ref[...].reshape(_H, 1),
                                       0.0)

    @pl.when(t == _NI + _NJ - 1)
    def _():
        Bt_scr[...] = jax.lax.dot_general(
            W3_ref[...], h2t_scr[...], _TDIMS,
            preferred_element_type=jnp.float32).astype(jnp.bfloat16)

    @pl.when(t >= _NI + _NJ)
    def _():
        cols = pl.ds((t - _NI - _NJ) * _JB, _JB)
        agg_t = jnp.dot(Bt_scr[...], Abig_scr[:, cols],
                        preferred_element_type=jnp.float32)
        x3t = agg_t + b3_ref[...].reshape(_H, 1)
        x1t = h1t_scr[:, cols]
        x2t = h2t_scr[:, cols]
        LWt = LWt_ref[...]
        logits = (jnp.dot(LWt[:, :_H], x1t,
                          preferred_element_type=jnp.float32)
                  + jnp.dot(LWt[:, _H:2 * _H], x2t,
                            preferred_element_type=jnp.float32)
                  + jnp.dot(LWt[:, 2 * _H:], x3t,
                            preferred_element_type=jnp.float32)
                  + lb_ref[...].reshape(_NCLS, 1))
        m = jnp.max(logits, axis=0, keepdims=True)
        s = logits - m
        lse = jnp.log(jnp.sum(jnp.exp(s), axis=0, keepdims=True))
        out_ref[...] = s - lse


def kernel(x, edge_index, W1, W2, W3, b1, b2, b3, lin_W, lin_b):
    n, d_in = x.shape
    A = edge_index

    full = lambda r, c: pl.BlockSpec((r, c), lambda t: (0, 0))
    vec = lambda m: pl.BlockSpec((m,), lambda t: (0,))
    out_t = pl.pallas_call(
        _gcn_kernel,
        grid=(_NI + 2 * _NJ,),
        in_specs=[
            pl.BlockSpec((_IB, _N),
                         lambda t: (jnp.minimum(t, _NI - 1), 0)),
            pl.BlockSpec((_IB, d_in),
                         lambda t: (jnp.minimum(t, _NI - 1), 0)),
            full(_H, d_in),
            full(_H, _H), full(_H, _H),
            vec(_H), vec(_H), vec(_H),
            full(_NCLS, 3 * _H),
            vec(_NCLS),
        ],
        out_specs=pl.BlockSpec(
            (_NCLS, _JB),
            lambda t: (0, jnp.maximum(t - _NI - _NJ, 0))),
        out_shape=jax.ShapeDtypeStruct((_NCLS, _N), jnp.float32),
        scratch_shapes=[
            pltpu.VMEM((_N, _N), jnp.bfloat16),
            pltpu.VMEM((_H, _N), jnp.float32),
            pltpu.VMEM((_H, _N), jnp.bfloat16),
            pltpu.VMEM((_H, _N), jnp.bfloat16),
            pltpu.VMEM((_H, _N), jnp.float32),
            pltpu.VMEM((_H, _N), jnp.float32),
        ],
    )(
        A, x, W1.T, W2, W3,
        b1, b2, b3,
        lin_W.T, lin_b,
    )
    return out_t.T


# speedup vs baseline: 1.1110x; 1.1110x over previous
import jax
import jax.numpy as jnp
from jax.experimental import pallas as pl
from jax.experimental.pallas import tpu as pltpu

_N = 4096
_H = 64
_NCLS = 10
_IB = 512
_NI = _N // _IB
_JB = 2048
_NJ = _N // _JB

_TDIMS = (((0,), (0,)), ((), ()))
_RTDIMS = (((1,), (1,)), ((), ()))


def _gcn_kernel(A_ref, x_ref, W1t_ref, W2_ref, W3_ref,
                b1_ref, b2_ref, b3_ref, LWt_ref, lb_ref,
                out_ref, Abig_scr, agg1_scr, Bt_scr, Ct_scr,
                h1t_scr, h2t_scr):
    t = pl.program_id(0)

    @pl.when(t < _NI)
    def _():
        rows = pl.ds(t * _IB, _IB)
        Abf = A_ref[...].astype(jnp.bfloat16)
        Abig_scr[rows, :] = Abf
        Bc = jax.lax.dot_general(
            W1t_ref[...], x_ref[...], _RTDIMS,
            preferred_element_type=jnp.float32)
        contrib = jnp.dot(Bc.astype(jnp.bfloat16), Abf,
                          preferred_element_type=jnp.float32)
        agg1_scr[...] = jnp.where(t == 0, contrib,
                                  agg1_scr[...] + contrib)

    @pl.when(t == _NI - 1)
    def _():
        x1t = jnp.maximum(agg1_scr[...] + b1_ref[...].reshape(_H, 1), 0.0)
        h1t_scr[...] = x1t
        Ct_scr[...] = jax.lax.dot_general(
            W2_ref[...], x1t, _TDIMS,
            preferred_element_type=jnp.float32).astype(jnp.bfloat16)

    @pl.when(jnp.logical_and(t >= _NI, t < _NI + _NJ))
    def _():
        cols = pl.ds((t - _NI) * _JB, _JB)
        agg_t = jnp.dot(Ct_scr[...], Abig_scr[:, cols],
                        preferred_element_type=jnp.float32)
        h2t_scr[:, cols] = jnp.maximum(agg_t + b2_ref[...].reshape(_H, 1),
                                       0.0)

    @pl.when(t == _NI + _NJ - 1)
    def _():
        Bt_scr[...] = jax.lax.dot_general(
            W3_ref[...], h2t_scr[...], _TDIMS,
            preferred_element_type=jnp.float32).astype(jnp.bfloat16)

    @pl.when(t >= _NI + _NJ)
    def _():
        cols = pl.ds((t - _NI - _NJ) * _JB, _JB)
        agg_t = jnp.dot(Bt_scr[...], Abig_scr[:, cols],
                        preferred_element_type=jnp.float32)
        x3t = agg_t + b3_ref[...].reshape(_H, 1)
        x1t = h1t_scr[:, cols]
        x2t = h2t_scr[:, cols]
        LWt = LWt_ref[...]
        logits = (jnp.dot(LWt[:, :_H], x1t,
                          preferred_element_type=jnp.float32)
                  + jnp.dot(LWt[:, _H:2 * _H], x2t,
                            preferred_element_type=jnp.float32)
                  + jnp.dot(LWt[:, 2 * _H:], x3t,
                            preferred_element_type=jnp.float32)
                  + lb_ref[...].reshape(_NCLS, 1))
        m = jnp.max(logits, axis=0, keepdims=True)
        s = logits - m
        lse = jnp.log(jnp.sum(jnp.exp(s), axis=0, keepdims=True))
        out_ref[...] = s - lse


def kernel(x, edge_index, W1, W2, W3, b1, b2, b3, lin_W, lin_b):
    n, d_in = x.shape
    A = edge_index

    full = lambda r, c: pl.BlockSpec((r, c), lambda t: (0, 0))
    vec = lambda m: pl.BlockSpec((m,), lambda t: (0,))
    out_t = pl.pallas_call(
        _gcn_kernel,
        grid=(_NI + 2 * _NJ,),
        in_specs=[
            pl.BlockSpec((_IB, _N),
                         lambda t: (jnp.minimum(t, _NI - 1), 0)),
            pl.BlockSpec((_IB, d_in),
                         lambda t: (jnp.minimum(t, _NI - 1), 0)),
            full(_H, d_in),
            full(_H, _H), full(_H, _H),
            vec(_H), vec(_H), vec(_H),
            full(_NCLS, 3 * _H),
            vec(_NCLS),
        ],
        out_specs=pl.BlockSpec(
            (_NCLS, _JB),
            lambda t: (0, jnp.maximum(t - _NI - _NJ, 0))),
        out_shape=jax.ShapeDtypeStruct((_NCLS, _N), jnp.float32),
        scratch_shapes=[
            pltpu.VMEM((_N, _N), jnp.bfloat16),
            pltpu.VMEM((_H, _N), jnp.float32),
            pltpu.VMEM((_H, _N), jnp.bfloat16),
            pltpu.VMEM((_H, _N), jnp.bfloat16),
            pltpu.VMEM((_H, _N), jnp.float32),
            pltpu.VMEM((_H, _N), jnp.float32),
        ],
    )(
        A, x, W1.T, W2, W3,
        b1, b2, b3,
        lin_W.T, lin_b,
    )
    return out_t.T
